# baseline (speedup 1.0000x reference)
import functools

import jax
import jax.numpy as jnp
from jax import lax
from jax.experimental import pallas as pl
from jax.experimental.pallas import tpu as pltpu
from jax.experimental.pallas import tpu_sc as plsc

_NC = 2
_NS = 16
_NW = _NC * _NS
_LANES = 16

_NP = 10240
_K = 128
_CPT = 80
_U = 4
_EPAD = _NW * _CPT * _K
_DUMP = _NP - 2


def _sc_degree(dst2d, np_nodes):
  mesh = plsc.VectorSubcoreMesh(core_axis_name="c", subcore_axis_name="s")
  rows_per_tile = np_nodes // _NS
  zrows = 16

  @functools.partial(
      pl.kernel,
      out_type=jax.ShapeDtypeStruct((_NC, np_nodes, _LANES), jnp.float32),
      mesh=mesh,
      scratch_types=[
          pltpu.VMEM((_CPT, _K), jnp.int32),
          pltpu.VMEM((_K, _LANES), jnp.float32),
          pltpu.VMEM((zrows, _LANES), jnp.float32),
          pltpu.VMEM_SHARED((np_nodes, _LANES), jnp.float32),
      ],
  )
  def deg_kernel(dst_hbm, out_hbm, idx_v, ones_v, zbuf_v, acc_sh):
    c = lax.axis_index("c")
    s = lax.axis_index("s")
    wid = c * _NS + s
    row0 = s * rows_per_tile

    for r in range(zrows):
      zbuf_v[r, :] = jnp.zeros((_LANES,), jnp.float32)
    for r in range(_K):
      ones_v[r, :] = jnp.ones((_LANES,), jnp.float32)

    pltpu.sync_copy(dst_hbm.at[pl.ds(wid * _CPT, _CPT)], idx_v)

    for q in range(rows_per_tile // zrows):
      pltpu.sync_copy(zbuf_v, acc_sh.at[pl.ds(row0 + q * zrows, zrows)])
    plsc.subcore_barrier()

    def body(j, carry):
      pltpu.sync_copy(ones_v, acc_sh.at[idx_v.at[j]], add=True)
      return carry

    lax.fori_loop(0, _CPT, body, 0)
    plsc.subcore_barrier()

    pltpu.sync_copy(
        acc_sh.at[pl.ds(row0, rows_per_tile)],
        out_hbm.at[c, pl.ds(row0, rows_per_tile)],
    )

  return deg_kernel(dst2d)


def _sc_scatter(g, src2d, dst2d):
  np_nodes, f = g.shape
  mesh = plsc.VectorSubcoreMesh(core_axis_name="c", subcore_axis_name="s")
  rows_per_tile = np_nodes // _NS
  zrows = 16
  sb = 16

  @functools.partial(
      pl.kernel,
      out_type=jax.ShapeDtypeStruct((_NC, np_nodes, f), jnp.float32),
      mesh=mesh,
      scratch_types=[
          pltpu.VMEM((sb, _K), jnp.int32),
          pltpu.VMEM((sb, _K), jnp.int32),
          [pltpu.VMEM((_K, f), jnp.float32) for _ in range(2)],
          pltpu.VMEM_SHARED((np_nodes, f), jnp.float32),
          [pltpu.SemaphoreType.DMA for _ in range(2)],
      ],
  )
  def scat_kernel(g_hbm, src_hbm, dst_hbm, out_hbm, idxs_v, idxd_v, rows_v,
                  acc_sh, sems):
    c = lax.axis_index("c")
    s = lax.axis_index("s")
    wid = c * _NS + s
    row0 = s * rows_per_tile

    for r in range(zrows):
      for q in range(f // _LANES):
        rows_v[0][r, pl.ds(q * _LANES, _LANES)] = jnp.zeros(
            (_LANES,), jnp.float32)
    for q in range(rows_per_tile // zrows):
      pltpu.sync_copy(rows_v[0].at[pl.ds(0, zrows)],
                      acc_sh.at[pl.ds(row0 + q * zrows, zrows)])
    plsc.subcore_barrier()

    def gather(q, buf, sem):
      return pltpu.async_copy(g_hbm.at[idxs_v.at[q]], buf, sem)

    def scat(q, buf):
      pltpu.sync_copy(buf, acc_sh.at[idxd_v.at[q]], add=True)

    def body(ob, carry):
      pltpu.sync_copy(src_hbm.at[pl.ds(wid * _CPT + ob * sb, sb)], idxs_v)
      pltpu.sync_copy(dst_hbm.at[pl.ds(wid * _CPT + ob * sb, sb)], idxd_v)
      for q in range(sb):
        gather(q, rows_v[0], sems[0]).wait()
        scat(q, rows_v[0])
      return carry

    lax.fori_loop(0, _CPT // sb, body, 0)
    plsc.subcore_barrier()

    pltpu.sync_copy(
        acc_sh.at[pl.ds(row0, rows_per_tile)],
        out_hbm.at[c, pl.ds(row0, rows_per_tile)],
    )

  return scat_kernel(g, src2d, dst2d)


_BM = 2000


def _tc_matmul(x, w):
  n, din = x.shape
  dout = w.shape[1]

  def body(x_ref, w_ref, o_ref):
    o_ref[...] = jnp.dot(x_ref[...], w_ref[...],
                         preferred_element_type=jnp.float32)

  return pl.pallas_call(
      body,
      grid=(n // _BM,),
      in_specs=[
          pl.BlockSpec((_BM, din), lambda i: (i, 0)),
          pl.BlockSpec((din, dout), lambda i: (0, 0)),
      ],
      out_specs=pl.BlockSpec((_BM, dout), lambda i: (i, 0)),
      out_shape=jax.ShapeDtypeStruct((n, dout), jnp.float32),
  )(x, w)


def _tc_prep(degp, h1):
  n, f = h1.shape

  def body(degp_ref, h1_ref, g1_ref, dinv_ref):
    deg = degp_ref[0, :, 0:1] + degp_ref[1, :, 0:1] + 1.0
    dv = lax.rsqrt(deg)
    dinv_ref[...] = dv
    g1_ref[...] = h1_ref[...] * dv

  return pl.pallas_call(
      body,
      grid=(n // _BM,),
      in_specs=[
          pl.BlockSpec((_NC, _BM, _LANES), lambda i: (0, i, 0)),
          pl.BlockSpec((_BM, f), lambda i: (i, 0)),
      ],
      out_specs=[
          pl.BlockSpec((_BM, f), lambda i: (i, 0)),
          pl.BlockSpec((_BM, 1), lambda i: (i, 0)),
      ],
      out_shape=[
          jax.ShapeDtypeStruct((_NP, f), jnp.float32),
          jax.ShapeDtypeStruct((n, 1), jnp.float32),
      ],
  )(degp, h1)


def _tc_mid(s1, g1, dinv, b1, w2):
  n = dinv.shape[0]
  f = g1.shape[1]
  dout = w2.shape[1]

  def body(s1_ref, g1_ref, dinv_ref, b1_ref, w2_ref, g2_ref):
    dv = dinv_ref[...]
    z = dv * (s1_ref[0] + s1_ref[1] + g1_ref[...]) + b1_ref[...]
    a = jnp.maximum(z, 0.0)
    h2 = jnp.dot(a, w2_ref[...], preferred_element_type=jnp.float32)
    g2_ref[...] = h2 * dv

  return pl.pallas_call(
      body,
      grid=(n // _BM,),
      in_specs=[
          pl.BlockSpec((_NC, _BM, f), lambda i: (0, i, 0)),
          pl.BlockSpec((_BM, f), lambda i: (i, 0)),
          pl.BlockSpec((_BM, 1), lambda i: (i, 0)),
          pl.BlockSpec((1, f), lambda i: (0, 0)),
          pl.BlockSpec((f, dout), lambda i: (0, 0)),
      ],
      out_specs=pl.BlockSpec((_BM, dout), lambda i: (i, 0)),
      out_shape=jax.ShapeDtypeStruct((_NP, dout), jnp.float32),
  )(s1, g1, dinv, b1, w2)


def _tc_final(s2, g2, dinv, b2):
  n = dinv.shape[0]
  fpad = g2.shape[1]
  f = b2.shape[1]

  def body(s2_ref, g2_ref, dinv_ref, b2_ref, z_ref, lsm_ref):
    comb = s2_ref[0, :, :f] + s2_ref[1, :, :f] + g2_ref[:, :f]
    z = dinv_ref[...] * comb + b2_ref[...]
    z_ref[...] = z
    m = jnp.max(z, axis=1, keepdims=True)
    lse = jnp.log(jnp.sum(jnp.exp(z - m), axis=1, keepdims=True)) + m
    lsm_ref[...] = z - lse

  return pl.pallas_call(
      body,
      grid=(n // _BM,),
      in_specs=[
          pl.BlockSpec((_NC, _BM, fpad), lambda i: (0, i, 0)),
          pl.BlockSpec((_BM, fpad), lambda i: (i, 0)),
          pl.BlockSpec((_BM, 1), lambda i: (i, 0)),
          pl.BlockSpec((1, f), lambda i: (0, 0)),
      ],
      out_specs=[
          pl.BlockSpec((_BM, f), lambda i: (i, 0)),
          pl.BlockSpec((_BM, f), lambda i: (i, 0)),
      ],
      out_shape=[
          jax.ShapeDtypeStruct((n, f), jnp.float32),
          jax.ShapeDtypeStruct((n, f), jnp.float32),
      ],
  )(s2, g2, dinv, b2)


@jax.jit
def kernel(x, edge_index, W1, b1, W2, b2):
  e = edge_index.shape[1]

  epad = _EPAD - e
  pad_dst = 10000 + jax.lax.rem(
      jnp.arange(epad, dtype=jnp.int32), jnp.int32(_NP - 10000))
  src2d = jnp.reshape(
      jnp.concatenate([edge_index[0], jnp.zeros((epad,), jnp.int32)]),
      (_EPAD // _K, _K))
  dst2d = jnp.reshape(
      jnp.concatenate([edge_index[1], pad_dst]),
      (_EPAD // _K, _K))

  w2p = jnp.pad(W2, ((0, 0), (0, 128 - W2.shape[1])))

  degp = _sc_degree(dst2d, _NP)
  h1 = _tc_matmul(x, W1)
  g1, dinv = _tc_prep(degp, h1)
  s1 = _sc_scatter(g1, src2d, dst2d)
  g2 = _tc_mid(s1, g1, dinv, jnp.reshape(b1, (1, -1)), w2p)
  s2 = _sc_scatter(g2, src2d, dst2d)
  z, lsm = _tc_final(s2, g2, dinv, jnp.reshape(b2, (1, -1)))
  return (z, lsm)


# speedup vs baseline: 3.9313x; 3.9313x over previous
import functools

import jax
import jax.numpy as jnp
from jax import lax
from jax.experimental import pallas as pl
from jax.experimental.pallas import tpu as pltpu
from jax.experimental.pallas import tpu_sc as plsc

_NC = 2
_NS = 16
_NW = _NC * _NS
_LANES = 16

_NP = 10240
_K = 128
_CPT = 80
_U = 4
_EPAD = _NW * _CPT * _K
_DUMP = _NP - 2


def _sc_degree(dst2d, np_nodes):
  mesh = plsc.VectorSubcoreMesh(core_axis_name="c", subcore_axis_name="s")
  rows_per_tile = np_nodes // _NS
  zrows = 16

  @functools.partial(
      pl.kernel,
      out_type=jax.ShapeDtypeStruct((_NC, np_nodes, _LANES), jnp.float32),
      mesh=mesh,
      scratch_types=[
          pltpu.VMEM((_CPT, _K), jnp.int32),
          pltpu.VMEM((_K, _LANES), jnp.float32),
          pltpu.VMEM((zrows, _LANES), jnp.float32),
          pltpu.VMEM_SHARED((np_nodes, _LANES), jnp.float32),
      ],
  )
  def deg_kernel(dst_hbm, out_hbm, idx_v, ones_v, zbuf_v, acc_sh):
    c = lax.axis_index("c")
    s = lax.axis_index("s")
    wid = c * _NS + s
    row0 = s * rows_per_tile

    for r in range(zrows):
      zbuf_v[r, :] = jnp.zeros((_LANES,), jnp.float32)
    for r in range(_K):
      ones_v[r, :] = jnp.ones((_LANES,), jnp.float32)

    pltpu.sync_copy(dst_hbm.at[pl.ds(wid * _CPT, _CPT)], idx_v)

    for q in range(rows_per_tile // zrows):
      pltpu.sync_copy(zbuf_v, acc_sh.at[pl.ds(row0 + q * zrows, zrows)])
    plsc.subcore_barrier()

    def body(j, carry):
      pltpu.sync_copy(ones_v, acc_sh.at[idx_v.at[j]], add=True)
      return carry

    lax.fori_loop(0, _CPT, body, 0)
    plsc.subcore_barrier()

    pltpu.sync_copy(
        acc_sh.at[pl.ds(row0, rows_per_tile)],
        out_hbm.at[c, pl.ds(row0, rows_per_tile)],
    )

  return deg_kernel(dst2d)


def _sc_scatter(g, src2d, dst2d):
  np_nodes, f = g.shape
  mesh = plsc.VectorSubcoreMesh(core_axis_name="c", subcore_axis_name="s")
  rows_per_tile = np_nodes // _NS
  zrows = 16
  sb = 16

  @functools.partial(
      pl.kernel,
      out_type=jax.ShapeDtypeStruct((_NC, np_nodes, f), jnp.float32),
      mesh=mesh,
      scratch_types=[
          pltpu.VMEM((sb, _K), jnp.int32),
          pltpu.VMEM((sb, _K), jnp.int32),
          [pltpu.VMEM((_K, f), jnp.float32) for _ in range(2)],
          pltpu.VMEM_SHARED((np_nodes, f), jnp.float32),
          [pltpu.SemaphoreType.DMA for _ in range(2)],
      ],
  )
  def scat_kernel(g_hbm, src_hbm, dst_hbm, out_hbm, idxs_v, idxd_v, rows_v,
                  acc_sh, sems):
    c = lax.axis_index("c")
    s = lax.axis_index("s")
    wid = c * _NS + s
    row0 = s * rows_per_tile

    for r in range(zrows):
      for q in range(f // _LANES):
        rows_v[0][r, pl.ds(q * _LANES, _LANES)] = jnp.zeros(
            (_LANES,), jnp.float32)
    for q in range(rows_per_tile // zrows):
      pltpu.sync_copy(rows_v[0].at[pl.ds(0, zrows)],
                      acc_sh.at[pl.ds(row0 + q * zrows, zrows)])
    plsc.subcore_barrier()

    def gather(q, buf, sem):
      return pltpu.async_copy(g_hbm.at[idxs_v.at[q]], buf, sem)

    def scat(q, buf):
      pltpu.sync_copy(buf, acc_sh.at[idxd_v.at[q]], add=True)

    def body(ob, carry):
      pltpu.sync_copy(src_hbm.at[pl.ds(wid * _CPT + ob * sb, sb)], idxs_v)
      pltpu.sync_copy(dst_hbm.at[pl.ds(wid * _CPT + ob * sb, sb)], idxd_v)
      desc = gather(0, rows_v[0], sems[0])
      for q in range(sb - 1):
        nxt = gather(q + 1, rows_v[(q + 1) % 2], sems[(q + 1) % 2])
        desc.wait()
        scat(q, rows_v[q % 2])
        desc = nxt
      desc.wait()
      scat(sb - 1, rows_v[(sb - 1) % 2])
      return carry

    lax.fori_loop(0, _CPT // sb, body, 0)
    plsc.subcore_barrier()

    pltpu.sync_copy(
        acc_sh.at[pl.ds(row0, rows_per_tile)],
        out_hbm.at[c, pl.ds(row0, rows_per_tile)],
    )

  return scat_kernel(g, src2d, dst2d)


_BM = 2000


def _tc_matmul(x, w):
  n, din = x.shape
  dout = w.shape[1]

  def body(x_ref, w_ref, o_ref):
    o_ref[...] = jnp.dot(x_ref[...], w_ref[...],
                         preferred_element_type=jnp.float32)

  return pl.pallas_call(
      body,
      grid=(n // _BM,),
      in_specs=[
          pl.BlockSpec((_BM, din), lambda i: (i, 0)),
          pl.BlockSpec((din, dout), lambda i: (0, 0)),
      ],
      out_specs=pl.BlockSpec((_BM, dout), lambda i: (i, 0)),
      out_shape=jax.ShapeDtypeStruct((n, dout), jnp.float32),
  )(x, w)


def _tc_prep(degp, h1):
  n, f = h1.shape

  def body(degp_ref, h1_ref, g1_ref, dinv_ref):
    deg = degp_ref[0, :, 0:1] + degp_ref[1, :, 0:1] + 1.0
    dv = lax.rsqrt(deg)
    dinv_ref[...] = dv
    g1_ref[...] = h1_ref[...] * dv

  return pl.pallas_call(
      body,
      grid=(n // _BM,),
      in_specs=[
          pl.BlockSpec((_NC, _BM, _LANES), lambda i: (0, i, 0)),
          pl.BlockSpec((_BM, f), lambda i: (i, 0)),
      ],
      out_specs=[
          pl.BlockSpec((_BM, f), lambda i: (i, 0)),
          pl.BlockSpec((_BM, 1), lambda i: (i, 0)),
      ],
      out_shape=[
          jax.ShapeDtypeStruct((_NP, f), jnp.float32),
          jax.ShapeDtypeStruct((n, 1), jnp.float32),
      ],
  )(degp, h1)


def _tc_mid(s1, g1, dinv, b1, w2):
  n = dinv.shape[0]
  f = g1.shape[1]
  dout = w2.shape[1]

  def body(s1_ref, g1_ref, dinv_ref, b1_ref, w2_ref, g2_ref):
    dv = dinv_ref[...]
    z = dv * (s1_ref[0] + s1_ref[1] + g1_ref[...]) + b1_ref[...]
    a = jnp.maximum(z, 0.0)
    h2 = jnp.dot(a, w2_ref[...], preferred_element_type=jnp.float32)
    g2_ref[...] = h2 * dv

  return pl.pallas_call(
      body,
      grid=(n // _BM,),
      in_specs=[
          pl.BlockSpec((_NC, _BM, f), lambda i: (0, i, 0)),
          pl.BlockSpec((_BM, f), lambda i: (i, 0)),
          pl.BlockSpec((_BM, 1), lambda i: (i, 0)),
          pl.BlockSpec((1, f), lambda i: (0, 0)),
          pl.BlockSpec((f, dout), lambda i: (0, 0)),
      ],
      out_specs=pl.BlockSpec((_BM, dout), lambda i: (i, 0)),
      out_shape=jax.ShapeDtypeStruct((_NP, dout), jnp.float32),
  )(s1, g1, dinv, b1, w2)


def _tc_final(s2, g2, dinv, b2):
  n = dinv.shape[0]
  fpad = g2.shape[1]
  f = b2.shape[1]

  def body(s2_ref, g2_ref, dinv_ref, b2_ref, z_ref, lsm_ref):
    comb = s2_ref[0, :, :f] + s2_ref[1, :, :f] + g2_ref[:, :f]
    z = dinv_ref[...] * comb + b2_ref[...]
    z_ref[...] = z
    m = jnp.max(z, axis=1, keepdims=True)
    lse = jnp.log(jnp.sum(jnp.exp(z - m), axis=1, keepdims=True)) + m
    lsm_ref[...] = z - lse

  return pl.pallas_call(
      body,
      grid=(n // _BM,),
      in_specs=[
          pl.BlockSpec((_NC, _BM, fpad), lambda i: (0, i, 0)),
          pl.BlockSpec((_BM, fpad), lambda i: (i, 0)),
          pl.BlockSpec((_BM, 1), lambda i: (i, 0)),
          pl.BlockSpec((1, f), lambda i: (0, 0)),
      ],
      out_specs=[
          pl.BlockSpec((_BM, f), lambda i: (i, 0)),
          pl.BlockSpec((_BM, f), lambda i: (i, 0)),
      ],
      out_shape=[
          jax.ShapeDtypeStruct((n, f), jnp.float32),
          jax.ShapeDtypeStruct((n, f), jnp.float32),
      ],
  )(s2, g2, dinv, b2)


@jax.jit
def kernel(x, edge_index, W1, b1, W2, b2):
  e = edge_index.shape[1]

  epad = _EPAD - e
  iota = jnp.arange(epad, dtype=jnp.int32)
  pad_dst = 10000 + jax.lax.rem(iota, jnp.int32(_NP - 10000))
  pad_src = jax.lax.rem(iota * 613, jnp.int32(10000))
  src2d = jnp.reshape(
      jnp.concatenate([edge_index[0], pad_src]),
      (_EPAD // _K, _K))
  dst2d = jnp.reshape(
      jnp.concatenate([edge_index[1], pad_dst]),
      (_EPAD // _K, _K))

  w2p = jnp.pad(W2, ((0, 0), (0, 128 - W2.shape[1])))

  degp = _sc_degree(dst2d, _NP)
  h1 = _tc_matmul(x, W1)
  g1, dinv = _tc_prep(degp, h1)
  s1 = _sc_scatter(g1, src2d, dst2d)
  g2 = _tc_mid(s1, g1, dinv, jnp.reshape(b1, (1, -1)), w2p)
  s2 = _sc_scatter(g2, src2d, dst2d)
  z, lsm = _tc_final(s2, g2, dinv, jnp.reshape(b2, (1, -1)))
  return (z, lsm)
